# baseline (speedup 1.0000x reference)
import functools

import jax
import jax.numpy as jnp
from jax import lax
from jax.experimental import pallas as pl
from jax.experimental.pallas import tpu as pltpu
from jax.experimental.pallas import tpu_sc as plsc

N = 10000
E = 320000
D_IN = 128
HID = 64
K_ORDER = 3
C_OUT = 40
ALPHA = 0.5
EPS = 1e-5

NC = 2
NS = 16
NW = NC * NS
CHUNK = 128
NCHUNK = 80
EPT = NCHUNK * CHUNK
E_MAIN = EPT * NW
N_PAD = 10112
RPT = N_PAD // NS
NB = 8
H = NB // 2
NCG = NCHUNK + H

_mesh = plsc.VectorSubcoreMesh(core_axis_name="c", subcore_axis_name="s")
_sc_params = pltpu.CompilerParams(use_tc_tiling_on_sc=False)


def _sc_deg_body(col_hbm, z16_hbm, ones_hbm, out0, out1,
                 accum, cidx, ones_v, *ssem):
    c = lax.axis_index("c")
    s = lax.axis_index("s")
    w = c * NS + s
    sl = pl.ds(s * RPT, RPT)
    pltpu.sync_copy(z16_hbm, accum.at[sl])
    pltpu.sync_copy(col_hbm.at[w], cidx)
    pltpu.sync_copy(ones_hbm, ones_v)
    plsc.subcore_barrier()

    def s_issue(ci, b):
        pltpu.async_copy(ones_v, accum.at[cidx.at[ci]], ssem[b], add=True)

    def s_wait(ci, b):
        pltpu.make_async_copy(ones_v, accum.at[cidx.at[ci]], ssem[b]).wait()

    for b in range(NB):
        s_issue(b, b)

    def body(i, carry):
        for b in range(NB):
            ci = NB * i + b
            s_wait(ci - NB, b)
            s_issue(ci, b)
        return carry

    lax.fori_loop(1, NCHUNK // NB, body, 0)
    for b in range(NB):
        s_wait(NCHUNK - NB + b, b)
    plsc.subcore_barrier()

    @pl.when(c == 0)
    def _():
        pltpu.sync_copy(accum.at[sl], out0.at[sl])

    @pl.when(c == 1)
    def _():
        pltpu.sync_copy(accum.at[sl], out1.at[sl])


_sc_deg = functools.partial(
    pl.kernel,
    mesh=_mesh,
    out_type=(
        jax.ShapeDtypeStruct((N_PAD, 16), jnp.float32),
        jax.ShapeDtypeStruct((N_PAD, 16), jnp.float32),
    ),
    scratch_types=[
        pltpu.VMEM_SHARED((N_PAD, 16), jnp.float32),
        pltpu.VMEM((NCG, CHUNK), jnp.int32),
        pltpu.VMEM((CHUNK, 16), jnp.float32),
    ] + [pltpu.SemaphoreType.DMA] * NB,
    compiler_params=_sc_params,
)(_sc_deg_body)


def _sc_hop_body(y_hbm, row_hbm, col_hbm, z64_hbm, out0, out1,
                 accum, ridx, cidx, bufs, *sems):
    gsem = sems[:NB]
    ssem = sems[NB:]
    c = lax.axis_index("c")
    s = lax.axis_index("s")
    w = c * NS + s
    sl = pl.ds(s * RPT, RPT)
    pltpu.sync_copy(z64_hbm, accum.at[sl])
    pltpu.sync_copy(row_hbm.at[w], ridx)
    pltpu.sync_copy(col_hbm.at[w], cidx)
    plsc.subcore_barrier()

    def g_issue(ci, b):
        pltpu.async_copy(y_hbm.at[ridx.at[ci]], bufs.at[b], gsem[b])

    def g_wait(ci, b):
        pltpu.make_async_copy(y_hbm.at[ridx.at[ci]], bufs.at[b],
                              gsem[b]).wait()

    def s_issue(ci, b):
        pltpu.async_copy(bufs.at[b], accum.at[cidx.at[ci]], ssem[b], add=True)

    def s_wait(ci, b):
        pltpu.make_async_copy(bufs.at[b], accum.at[cidx.at[ci]],
                              ssem[b]).wait()

    g_issue(0, 0)
    g_issue(1, 1)
    g_issue(2, 2)

    def body(i, carry):
        for b in range(4):
            ci = 4 * i + b
            g_wait(ci, b)
            g_issue(ci + 3, (b + 3) % 4)
            pltpu.sync_copy(bufs.at[b], accum.at[cidx.at[ci]], add=True)
        return carry

    lax.fori_loop(0, NCHUNK // 4, body, 0)
    g_wait(NCHUNK, 0)
    g_wait(NCHUNK + 1, 1)
    g_wait(NCHUNK + 2, 2)
    plsc.subcore_barrier()

    @pl.when(c == 0)
    def _():
        pltpu.sync_copy(accum.at[sl], out0.at[sl])

    @pl.when(c == 1)
    def _():
        pltpu.sync_copy(accum.at[sl], out1.at[sl])


_sc_hop = functools.partial(
    pl.kernel,
    mesh=_mesh,
    out_type=(
        jax.ShapeDtypeStruct((N_PAD, HID), jnp.float32),
        jax.ShapeDtypeStruct((N_PAD, HID), jnp.float32),
    ),
    scratch_types=[
        pltpu.VMEM_SHARED((N_PAD, HID), jnp.float32),
        pltpu.VMEM((NCG, CHUNK), jnp.int32),
        pltpu.VMEM((NCG, CHUNK), jnp.int32),
        pltpu.VMEM((NB, CHUNK, HID), jnp.float32),
    ] + [pltpu.SemaphoreType.DMA] * (2 * NB),
    compiler_params=_sc_params,
)(_sc_hop_body)


def _bn_relu(hh, g, be):
    m = jnp.mean(hh, axis=0, keepdims=True)
    hc = hh - m
    v = jnp.mean(hc * hc, axis=0, keepdims=True)
    return jnp.maximum(g * hc / jnp.sqrt(v + EPS) + be, 0.0)


def _tc_in_body(x_r, d0_r, d1_r, W0_r, b0_r, g0_r, be0_r, h_o, y_o, dv_o):
    deg = d0_r[...][:N, 0:1] + d1_r[...][:N, 0:1]
    dinv = jnp.where(deg > 0.0, 1.0 / jnp.sqrt(deg), 0.0)
    dv = jnp.broadcast_to(dinv, (N, HID))
    hh = jnp.dot(x_r[...], W0_r[...], preferred_element_type=jnp.float32) \
        + b0_r[...]
    h = _bn_relu(hh, g0_r[...], be0_r[...])
    h_o[...] = h
    y_o[...] = dv * h
    dv_o[...] = dv


def _tc_attn_body(h_r, Wq_r, bq_r, Wk_r, bk_r, a_o):
    h = h_r[...]
    q = jnp.dot(h, Wq_r[...], preferred_element_type=jnp.float32) + bq_r[...]
    k = jnp.dot(h, Wk_r[...], preferred_element_type=jnp.float32) + bk_r[...]
    q = q / jnp.sqrt(jnp.sum(q * q, axis=1, keepdims=True))
    k = k / jnp.sqrt(jnp.sum(k * k, axis=1, keepdims=True))
    kvs = lax.dot_general(k, h, (((0,), (0,)), ((), ())),
                          preferred_element_type=jnp.float32)
    num = jnp.dot(q, kvs, preferred_element_type=jnp.float32) + \
        jnp.sum(h, axis=0, keepdims=True)
    ksum = jnp.sum(k, axis=0, keepdims=True)
    den = jnp.sum(q * ksum, axis=1, keepdims=True) + float(N)
    a_o[...] = num / den


def _tc_mid_body(p0_r, p1_r, dv_r, x_o, y_o):
    dv = dv_r[...]
    t = p0_r[...][:N] + p1_r[...][:N]
    xk = dv * t
    x_o[...] = xk
    y_o[...] = dv * xk


def _tc_upd_body(h_r, a_r, x1_r, x2_r, x3_r, dv_r, Wf_r, bf_r, g_r, be_r,
                 h_o, y_o):
    h = h_r[...]
    Wf = Wf_r[...]
    hh = (jnp.dot(a_r[...], Wf[0:HID], preferred_element_type=jnp.float32)
          + jnp.dot(h, Wf[HID:2 * HID], preferred_element_type=jnp.float32)
          + jnp.dot(x1_r[...], Wf[2 * HID:3 * HID],
                    preferred_element_type=jnp.float32)
          + jnp.dot(x2_r[...], Wf[3 * HID:4 * HID],
                    preferred_element_type=jnp.float32)
          + jnp.dot(x3_r[...], Wf[4 * HID:5 * HID],
                    preferred_element_type=jnp.float32)
          + bf_r[...])
    hn = _bn_relu(hh, g_r[...], be_r[...])
    h2 = ALPHA * hn + (1.0 - ALPHA) * h
    h_o[...] = h2
    y_o[...] = dv_r[...] * h2


def _tc_head_body(h_r, Wout_r, bout_r, o_o):
    o_o[...] = jnp.dot(h_r[...], Wout_r[...],
                       preferred_element_type=jnp.float32) + bout_r[...]


def _tc(body, out_shapes, *args):
    return pl.pallas_call(body, out_shape=out_shapes)(*args)


_NH = jax.ShapeDtypeStruct((N, HID), jnp.float32)


def kernel(x, edge_index, W0, b0, g0, be0, Wq0, bq0, Wk0, bk0, Wf0, bf0,
           g1, be1, Wq1, bq1, Wk1, bk1, Wf1, bf1, g2, be2, Wout, bout):
    ei = edge_index.astype(jnp.int32)
    row, col = ei[0], ei[1]
    pad = E_MAIN - E
    row_m = jnp.concatenate([row, jnp.zeros((pad,), jnp.int32)])
    col_m = jnp.concatenate([col, jnp.full((pad,), N, jnp.int32)])
    row3d = jnp.concatenate(
        [row_m.reshape(NW, NCHUNK, CHUNK),
         jnp.zeros((NW, H, CHUNK), jnp.int32)], axis=1)
    col3d = jnp.concatenate(
        [col_m.reshape(NW, NCHUNK, CHUNK),
         jnp.full((NW, H, CHUNK), N, jnp.int32)], axis=1)
    z16 = jnp.zeros((RPT, 16), jnp.float32)
    z64 = jnp.zeros((RPT, HID), jnp.float32)
    ones16 = jnp.ones((CHUNK, 16), jnp.float32)

    d0, d1 = _sc_deg(col3d, z16, ones16)
    h, y, dv = _tc(_tc_in_body, (_NH, _NH, _NH),
                   x, d0, d1, W0, b0, g0, be0)
    a = _tc(_tc_attn_body, _NH, h, Wq0, bq0, Wk0, bk0)

    out = None
    for (Wq, bq, Wk, bk, Wf, bf, g, be, last) in (
            (Wq1, bq1, Wk1, bk1, Wf0, bf0, g1, be1, False),
            (None, None, None, None, Wf1, bf1, g2, be2, True)):
        xs = []
        for _hop in range(K_ORDER):
            p0, p1 = _sc_hop(y, row3d, col3d, z64)
            xk, y = _tc(_tc_mid_body, (_NH, _NH), p0, p1, dv)
            xs.append(xk)
        h, y = _tc(_tc_upd_body, (_NH, _NH),
                   h, a, xs[0], xs[1], xs[2], dv, Wf, bf, g, be)
        if not last:
            a = _tc(_tc_attn_body, _NH, h, Wq, bq, Wk, bk)
        else:
            out = _tc(_tc_head_body,
                      jax.ShapeDtypeStruct((N, C_OUT), jnp.float32),
                      h, Wout, bout)
    return out


# speedup vs baseline: 1.3622x; 1.3622x over previous
import functools

import jax
import jax.numpy as jnp
from jax import lax
from jax.experimental import pallas as pl
from jax.experimental.pallas import tpu as pltpu
from jax.experimental.pallas import tpu_sc as plsc

N = 10000
E = 320000
D_IN = 128
HID = 64
K_ORDER = 3
C_OUT = 40
ALPHA = 0.5
EPS = 1e-5

NC = 2
NS = 16
NW = NC * NS
CHUNK = 128
NCHUNK = 80
EPT = NCHUNK * CHUNK
E_MAIN = EPT * NW
N_PAD = 10112
RPT = N_PAD // NS
NB = 8
H = NB // 2
NCG = NCHUNK + H

_mesh = plsc.VectorSubcoreMesh(core_axis_name="c", subcore_axis_name="s")
_sc_params = pltpu.CompilerParams(use_tc_tiling_on_sc=False)


def _sc_deg_body(col_hbm, z16_hbm, ones_hbm, out0, out1,
                 accum, cidx, ones_v, *ssem):
    c = lax.axis_index("c")
    s = lax.axis_index("s")
    w = c * NS + s
    sl = pl.ds(s * RPT, RPT)
    pltpu.sync_copy(z16_hbm, accum.at[sl])
    pltpu.sync_copy(col_hbm.at[w], cidx)
    pltpu.sync_copy(ones_hbm, ones_v)
    plsc.subcore_barrier()

    def s_issue(ci, b):
        pltpu.async_copy(ones_v, accum.at[cidx.at[ci]], ssem[b], add=True)

    def s_wait(ci, b):
        pltpu.make_async_copy(ones_v, accum.at[cidx.at[ci]], ssem[b]).wait()

    for b in range(NB):
        s_issue(b, b)

    def body(i, carry):
        for b in range(NB):
            ci = NB * i + b
            s_wait(ci - NB, b)
            s_issue(ci, b)
        return carry

    lax.fori_loop(1, NCHUNK // NB, body, 0)
    for b in range(NB):
        s_wait(NCHUNK - NB + b, b)
    plsc.subcore_barrier()

    @pl.when(c == 0)
    def _():
        pltpu.sync_copy(accum.at[sl], out0.at[sl])

    @pl.when(c == 1)
    def _():
        pltpu.sync_copy(accum.at[sl], out1.at[sl])


_sc_deg = functools.partial(
    pl.kernel,
    mesh=_mesh,
    out_type=(
        jax.ShapeDtypeStruct((N_PAD, 16), jnp.float32),
        jax.ShapeDtypeStruct((N_PAD, 16), jnp.float32),
    ),
    scratch_types=[
        pltpu.VMEM_SHARED((N_PAD, 16), jnp.float32),
        pltpu.VMEM((NCG, CHUNK), jnp.int32),
        pltpu.VMEM((CHUNK, 16), jnp.float32),
    ] + [pltpu.SemaphoreType.DMA] * NB,
    compiler_params=_sc_params,
)(_sc_deg_body)


def _sc_hop_body(y_hbm, row_hbm, col_hbm, z64_hbm, out0, out1,
                 accum, ridx, cidx, bufs, *sems):
    gsem = sems[:NB]
    ssem = sems[NB:]
    c = lax.axis_index("c")
    s = lax.axis_index("s")
    w = c * NS + s
    sl = pl.ds(s * RPT, RPT)
    pltpu.sync_copy(z64_hbm, accum.at[sl])
    pltpu.sync_copy(row_hbm.at[w], ridx)
    pltpu.sync_copy(col_hbm.at[w], cidx)
    plsc.subcore_barrier()

    def g_issue(ci, b):
        pltpu.async_copy(y_hbm.at[ridx.at[ci]], bufs.at[b], gsem[b])

    def g_wait(ci, b):
        pltpu.make_async_copy(y_hbm.at[ridx.at[ci]], bufs.at[b],
                              gsem[b]).wait()

    def s_issue(ci, b):
        pltpu.async_copy(bufs.at[b], accum.at[cidx.at[ci]], ssem[b], add=True)

    def s_wait(ci, b):
        pltpu.make_async_copy(bufs.at[b], accum.at[cidx.at[ci]],
                              ssem[b]).wait()

    g_issue(0, 0)

    def body(i, carry):
        ci = 2 * i
        g_wait(ci, 0)
        g_issue(ci + 1, 1)
        pltpu.sync_copy(bufs.at[0], accum.at[cidx.at[ci]], add=True)
        g_wait(ci + 1, 1)
        g_issue(ci + 2, 0)
        pltpu.sync_copy(bufs.at[1], accum.at[cidx.at[ci + 1]], add=True)
        return carry

    lax.fori_loop(0, NCHUNK // 2, body, 0)
    g_wait(NCHUNK, 0)
    plsc.subcore_barrier()

    @pl.when(c == 0)
    def _():
        pltpu.sync_copy(accum.at[sl], out0.at[sl])

    @pl.when(c == 1)
    def _():
        pltpu.sync_copy(accum.at[sl], out1.at[sl])


_sc_hop = functools.partial(
    pl.kernel,
    mesh=_mesh,
    out_type=(
        jax.ShapeDtypeStruct((N_PAD, HID), jnp.float32),
        jax.ShapeDtypeStruct((N_PAD, HID), jnp.float32),
    ),
    scratch_types=[
        pltpu.VMEM_SHARED((N_PAD, HID), jnp.float32),
        pltpu.VMEM((NCG, CHUNK), jnp.int32),
        pltpu.VMEM((NCG, CHUNK), jnp.int32),
        pltpu.VMEM((NB, CHUNK, HID), jnp.float32),
    ] + [pltpu.SemaphoreType.DMA] * (2 * NB),
    compiler_params=_sc_params,
)(_sc_hop_body)


def _bn_relu(hh, g, be):
    m = jnp.mean(hh, axis=0, keepdims=True)
    hc = hh - m
    v = jnp.mean(hc * hc, axis=0, keepdims=True)
    return jnp.maximum(g * hc / jnp.sqrt(v + EPS) + be, 0.0)


def _tc_in_body(x_r, d0_r, d1_r, W0_r, b0_r, g0_r, be0_r, h_o, y_o, dv_o):
    deg = d0_r[...][:N, 0:1] + d1_r[...][:N, 0:1]
    dinv = jnp.where(deg > 0.0, 1.0 / jnp.sqrt(deg), 0.0)
    dv = jnp.broadcast_to(dinv, (N, HID))
    hh = jnp.dot(x_r[...], W0_r[...], preferred_element_type=jnp.float32) \
        + b0_r[...]
    h = _bn_relu(hh, g0_r[...], be0_r[...])
    h_o[...] = h
    y_o[...] = dv * h
    dv_o[...] = dv


def _tc_attn_body(h_r, Wq_r, bq_r, Wk_r, bk_r, a_o):
    h = h_r[...]
    q = jnp.dot(h, Wq_r[...], preferred_element_type=jnp.float32) + bq_r[...]
    k = jnp.dot(h, Wk_r[...], preferred_element_type=jnp.float32) + bk_r[...]
    q = q / jnp.sqrt(jnp.sum(q * q, axis=1, keepdims=True))
    k = k / jnp.sqrt(jnp.sum(k * k, axis=1, keepdims=True))
    kvs = lax.dot_general(k, h, (((0,), (0,)), ((), ())),
                          preferred_element_type=jnp.float32)
    num = jnp.dot(q, kvs, preferred_element_type=jnp.float32) + \
        jnp.sum(h, axis=0, keepdims=True)
    ksum = jnp.sum(k, axis=0, keepdims=True)
    den = jnp.sum(q * ksum, axis=1, keepdims=True) + float(N)
    a_o[...] = num / den


def _tc_mid_body(p0_r, p1_r, dv_r, x_o, y_o):
    dv = dv_r[...]
    t = p0_r[...][:N] + p1_r[...][:N]
    xk = dv * t
    x_o[...] = xk
    y_o[...] = dv * xk


def _tc_upd_body(h_r, a_r, x1_r, x2_r, x3_r, dv_r, Wf_r, bf_r, g_r, be_r,
                 h_o, y_o):
    h = h_r[...]
    Wf = Wf_r[...]
    hh = (jnp.dot(a_r[...], Wf[0:HID], preferred_element_type=jnp.float32)
          + jnp.dot(h, Wf[HID:2 * HID], preferred_element_type=jnp.float32)
          + jnp.dot(x1_r[...], Wf[2 * HID:3 * HID],
                    preferred_element_type=jnp.float32)
          + jnp.dot(x2_r[...], Wf[3 * HID:4 * HID],
                    preferred_element_type=jnp.float32)
          + jnp.dot(x3_r[...], Wf[4 * HID:5 * HID],
                    preferred_element_type=jnp.float32)
          + bf_r[...])
    hn = _bn_relu(hh, g_r[...], be_r[...])
    h2 = ALPHA * hn + (1.0 - ALPHA) * h
    h_o[...] = h2
    y_o[...] = dv_r[...] * h2


def _tc_head_body(h_r, Wout_r, bout_r, o_o):
    o_o[...] = jnp.dot(h_r[...], Wout_r[...],
                       preferred_element_type=jnp.float32) + bout_r[...]


def _tc(body, out_shapes, *args):
    return pl.pallas_call(body, out_shape=out_shapes)(*args)


_NH = jax.ShapeDtypeStruct((N, HID), jnp.float32)


def kernel(x, edge_index, W0, b0, g0, be0, Wq0, bq0, Wk0, bk0, Wf0, bf0,
           g1, be1, Wq1, bq1, Wk1, bk1, Wf1, bf1, g2, be2, Wout, bout):
    ei = edge_index.astype(jnp.int32)
    row, col = ei[0], ei[1]
    pad = E_MAIN - E
    row_m = jnp.concatenate([row, jnp.zeros((pad,), jnp.int32)])
    col_m = jnp.concatenate([col, jnp.full((pad,), N, jnp.int32)])
    row3d = jnp.concatenate(
        [row_m.reshape(NW, NCHUNK, CHUNK),
         jnp.zeros((NW, H, CHUNK), jnp.int32)], axis=1)
    col3d = jnp.concatenate(
        [col_m.reshape(NW, NCHUNK, CHUNK),
         jnp.full((NW, H, CHUNK), N, jnp.int32)], axis=1)
    z16 = jnp.zeros((RPT, 16), jnp.float32)
    z64 = jnp.zeros((RPT, HID), jnp.float32)
    ones16 = jnp.ones((CHUNK, 16), jnp.float32)

    d0, d1 = _sc_deg(col3d, z16, ones16)
    h, y, dv = _tc(_tc_in_body, (_NH, _NH, _NH),
                   x, d0, d1, W0, b0, g0, be0)
    a = _tc(_tc_attn_body, _NH, h, Wq0, bq0, Wk0, bk0)

    out = None
    for (Wq, bq, Wk, bk, Wf, bf, g, be, last) in (
            (Wq1, bq1, Wk1, bk1, Wf0, bf0, g1, be1, False),
            (None, None, None, None, Wf1, bf1, g2, be2, True)):
        xs = []
        for _hop in range(K_ORDER):
            p0, p1 = _sc_hop(y, row3d, col3d, z64)
            xk, y = _tc(_tc_mid_body, (_NH, _NH), p0, p1, dv)
            xs.append(xk)
        h, y = _tc(_tc_upd_body, (_NH, _NH),
                   h, a, xs[0], xs[1], xs[2], dv, Wf, bf, g, be)
        if not last:
            a = _tc(_tc_attn_body, _NH, h, Wq, bq, Wk, bk)
        else:
            out = _tc(_tc_head_body,
                      jax.ShapeDtypeStruct((N, C_OUT), jnp.float32),
                      h, Wout, bout)
    return out
